# baseline (speedup 1.0000x reference)
import jax
import jax.numpy as jnp
from jax import lax
from jax.experimental import pallas as pl
from jax.experimental.pallas import tpu as pltpu
from jax.experimental.pallas import tpu_sc as plsc

B, S, D = 4, 4096, 2048
R = B * S
NC, NS, L = 2, 16, 16
NW = NC * NS
RSC = 8192
RW = RSC // NW
NBUF = 2
BLK = 512


def _sc_body(emb_hbm, uidx_hbm, midx_hbm, cnt_hbm, mrows_hbm, out_hbm,
             uidxv, midxv, cntv, stage0, stage1, mrows_v,
             gsem0, gsem1, ssem0, ssem1, msem):
    wid = lax.axis_index("s") * NC + lax.axis_index("c")
    base = wid * RW
    pltpu.sync_copy(cnt_hbm.at[pl.ds(wid * L, L)], cntv)
    pltpu.sync_copy(uidx_hbm.at[pl.ds(base, RW)], uidxv)
    pltpu.sync_copy(midx_hbm.at[pl.ds(base, RW)], midxv)
    pltpu.sync_copy(mrows_hbm, mrows_v)
    cv = cntv[...]
    ncu = cv[0]
    ncm = cv[1]
    stages = (stage0, stage1)
    gsems = (gsem0, gsem1)
    ssems = (ssem0, ssem1)

    def mfire(j, c):
        vi = midxv[pl.ds(j * L, L)]
        pltpu.async_copy(mrows_v, out_hbm.at[vi], msem)
        return c

    lax.fori_loop(0, ncm, mfire, jnp.int32(0))

    for b in range(NBUF):
        @pl.when(b < ncu)
        def _(b=b):
            vi = uidxv[pl.ds(b * L, L)]
            pltpu.async_copy(emb_hbm.at[vi], stages[b], gsems[b])

    def obody(t, c):
        for b in range(NBUF):
            j = NBUF * t + b

            @pl.when(j < ncu)
            def _(b=b, j=j):
                vi = uidxv[pl.ds(j * L, L)]
                pltpu.make_async_copy(
                    emb_hbm.at[vi], stages[b], gsems[b]).wait()
                pltpu.async_copy(
                    stages[b], out_hbm.at[vi], ssems[b]).wait()

                @pl.when(j + NBUF < ncu)
                def _():
                    vi2 = uidxv[pl.ds((j + NBUF) * L, L)]
                    pltpu.async_copy(emb_hbm.at[vi2], stages[b], gsems[b])
        return c

    lax.fori_loop(0, (ncu + NBUF - 1) // NBUF, obody, jnp.int32(0))

    def mdrain(j, c):
        pltpu.make_async_copy(mrows_v, out_hbm.at[midxv[pl.ds(0, L)]],
                              msem).wait()
        return c

    lax.fori_loop(0, ncm, mdrain, jnp.int32(0))


_mesh = plsc.VectorSubcoreMesh(core_axis_name="c", subcore_axis_name="s")

_sc_call = pl.kernel(
    _sc_body,
    mesh=_mesh,
    out_type=jax.ShapeDtypeStruct((RSC, D), jnp.float32),
    scratch_types=[
        pltpu.VMEM((RW,), jnp.int32),
        pltpu.VMEM((RW,), jnp.int32),
        pltpu.VMEM((L,), jnp.int32),
        pltpu.VMEM((L, D), jnp.float32),
        pltpu.VMEM((L, D), jnp.float32),
        pltpu.VMEM((L, D), jnp.float32),
        pltpu.SemaphoreType.DMA,
        pltpu.SemaphoreType.DMA,
        pltpu.SemaphoreType.DMA,
        pltpu.SemaphoreType.DMA,
        pltpu.SemaphoreType.DMA,
    ],
)


def _tc_body(mask_ref, mte_ref, emb_ref, out_ref):
    out_ref[...] = jnp.where(mask_ref[...] != 0, mte_ref[...], emb_ref[...])


def kernel(embeddings, token_mask, mask_token_embed):
    emb = embeddings.reshape(R, D)
    maskf = token_mask.reshape(R).astype(jnp.int32)

    mask2 = maskf[:RSC].reshape(NW, RW)
    notm = 1 - mask2
    nu = jnp.sum(notm, axis=1, keepdims=True)
    nm = RW - nu
    posu = jnp.cumsum(notm, axis=1) - 1
    r = jnp.arange(RW, dtype=jnp.int32)[None, :]
    w = jnp.arange(NW, dtype=jnp.int32)[:, None]
    rows = w * RW + r
    dest_u = jnp.where(notm == 1, posu, nu + (r - posu - 1))
    k3 = r.reshape(1, 1, RW)
    perm_u = jnp.sum(
        jnp.where(dest_u[:, :, None] == k3, rows[:, :, None], 0), axis=1)
    rev = perm_u[:, ::-1]
    uidx = jnp.where(r < nu, perm_u, perm_u[:, :1]).reshape(RSC)
    midx = jnp.where(r < nm, rev, rev[:, :1]).reshape(RSC)
    ncu = (nu[:, 0] + L - 1) // L
    ncm = (nm[:, 0] + L - 1) // L
    cnt = jnp.zeros((NW, L), jnp.int32)
    cnt = cnt.at[:, 0].set(ncu).at[:, 1].set(ncm).reshape(NW * L)
    mrows = jnp.broadcast_to(mask_token_embed.reshape(1, D), (L, D))
    sc_out = _sc_call(emb, uidx, midx, cnt, mrows)

    nblk = (R - RSC) // BLK
    off = RSC // BLK
    tc_out = pl.pallas_call(
        _tc_body,
        grid=(nblk,),
        in_specs=[
            pl.BlockSpec((BLK, 1), lambda i: (i + off, 0)),
            pl.BlockSpec((1, D), lambda i: (0, 0)),
            pl.BlockSpec((BLK, D), lambda i: (i + off, 0)),
        ],
        out_specs=pl.BlockSpec((BLK, D), lambda i: (i, 0)),
        out_shape=jax.ShapeDtypeStruct((R - RSC, D), jnp.float32),
        compiler_params=pltpu.CompilerParams(
            dimension_semantics=("arbitrary",),
        ),
    )(maskf.reshape(R, 1), mask_token_embed.reshape(1, D), emb)

    out = jnp.concatenate([sc_out, tc_out], axis=0)
    return out.reshape(B, S, D)


# speedup vs baseline: 1.8059x; 1.8059x over previous
import jax
import jax.numpy as jnp
from jax import lax
from jax.experimental import pallas as pl
from jax.experimental.pallas import tpu as pltpu
from jax.experimental.pallas import tpu_sc as plsc

B, S, D = 4, 4096, 2048
R = B * S
NC, NS, L = 2, 16, 16
NW = NC * NS
RW = R // NW
NBUF = 2


def _sc_body(emb_hbm, uidx_hbm, midx_hbm, cnt_hbm, mrows_hbm, out_hbm,
             uidxv, midxv, cntv, stage0, stage1, mrows_v,
             gsem0, gsem1, ssem0, ssem1, msem):
    wid = lax.axis_index("s") * NC + lax.axis_index("c")
    base = wid * RW
    pltpu.sync_copy(cnt_hbm.at[pl.ds(wid * L, L)], cntv)
    pltpu.sync_copy(uidx_hbm.at[pl.ds(base, RW)], uidxv)
    pltpu.sync_copy(midx_hbm.at[pl.ds(base, RW)], midxv)
    pltpu.sync_copy(mrows_hbm, mrows_v)
    cv = cntv[...]
    ncu = cv[0]
    ncm = cv[1]
    stages = (stage0, stage1)
    gsems = (gsem0, gsem1)
    ssems = (ssem0, ssem1)

    for b in range(NBUF):
        @pl.when(b < ncu)
        def _(b=b):
            vi = uidxv[pl.ds(b * L, L)]
            pltpu.async_copy(emb_hbm.at[vi], stages[b], gsems[b])

    def mfire(j, c):
        vi = midxv[pl.ds(j * L, L)]
        pltpu.async_copy(mrows_v, out_hbm.at[vi], msem)
        return c

    lax.fori_loop(0, ncm, mfire, jnp.int32(0))

    def obody(t, c):
        for b in range(NBUF):
            j = NBUF * t + b

            @pl.when(j < ncu)
            def _(b=b, j=j):
                vi = uidxv[pl.ds(j * L, L)]
                pltpu.make_async_copy(
                    emb_hbm.at[vi], stages[b], gsems[b]).wait()
                pltpu.async_copy(
                    stages[b], out_hbm.at[vi], ssems[b]).wait()

                @pl.when(j + NBUF < ncu)
                def _():
                    vi2 = uidxv[pl.ds((j + NBUF) * L, L)]
                    pltpu.async_copy(emb_hbm.at[vi2], stages[b], gsems[b])
        return c

    lax.fori_loop(0, (ncu + NBUF - 1) // NBUF, obody, jnp.int32(0))

    def mdrain(j, c):
        pltpu.make_async_copy(mrows_v, out_hbm.at[midxv[pl.ds(0, L)]],
                              msem).wait()
        return c

    lax.fori_loop(0, ncm, mdrain, jnp.int32(0))


_mesh = plsc.VectorSubcoreMesh(core_axis_name="c", subcore_axis_name="s")

_sc_call = pl.kernel(
    _sc_body,
    mesh=_mesh,
    out_type=jax.ShapeDtypeStruct((R, D), jnp.float32),
    scratch_types=[
        pltpu.VMEM((RW,), jnp.int32),
        pltpu.VMEM((RW,), jnp.int32),
        pltpu.VMEM((L,), jnp.int32),
        pltpu.VMEM((L, D), jnp.float32),
        pltpu.VMEM((L, D), jnp.float32),
        pltpu.VMEM((L, D), jnp.float32),
        pltpu.SemaphoreType.DMA,
        pltpu.SemaphoreType.DMA,
        pltpu.SemaphoreType.DMA,
        pltpu.SemaphoreType.DMA,
        pltpu.SemaphoreType.DMA,
    ],
)


def kernel(embeddings, token_mask, mask_token_embed):
    emb = embeddings.reshape(R, D)
    mask2 = token_mask.reshape(NW, RW).astype(jnp.int32)
    notm = 1 - mask2
    nu = jnp.sum(notm, axis=1, keepdims=True)
    nm = RW - nu
    posu = jnp.cumsum(notm, axis=1) - 1
    r = jnp.arange(RW, dtype=jnp.int32)[None, :]
    w = jnp.arange(NW, dtype=jnp.int32)[:, None]
    rows = w * RW + r
    dest_u = jnp.where(notm == 1, posu, nu + (r - posu - 1))
    k3 = r.reshape(1, 1, RW)
    perm_u = jnp.sum(
        jnp.where(dest_u[:, :, None] == k3, rows[:, :, None], 0), axis=1)
    rev = perm_u[:, ::-1]
    uidx = jnp.where(r < nu, perm_u, perm_u[:, :1]).reshape(R)
    midx = jnp.where(r < nm, rev, rev[:, :1]).reshape(R)
    ncu = (nu[:, 0] + L - 1) // L
    ncm = (nm[:, 0] + L - 1) // L
    cnt = jnp.zeros((NW, L), jnp.int32)
    cnt = cnt.at[:, 0].set(ncu).at[:, 1].set(ncm).reshape(NW * L)
    mrows = jnp.broadcast_to(mask_token_embed.reshape(1, D), (L, D))
    out = _sc_call(emb, uidx, midx, cnt, mrows)
    return out.reshape(B, S, D)
